# Initial kernel scaffold; baseline (speedup 1.0000x reference)
import jax
import jax.numpy as jnp
from jax.experimental import pallas as pl


def kernel(target, permutation):
    raise NotImplementedError("write your pallas kernel here")


# speedup vs baseline: 1.2502x; 1.2502x over previous
import functools

import jax
import jax.numpy as jnp
from jax import lax
from jax.experimental import pallas as pl
from jax.experimental.pallas import tpu as pltpu
from jax.experimental.pallas import tpu_sc as plsc

_LATENT = 128
_BATCH = 16384
_NC = 2
_NS = 16
_L = 16
_NW = _NC * _NS
_ROWS_PER_W = _BATCH // _NW
_CHUNK = 256
_NCHUNK = _ROWS_PER_W // _CHUNK
_GROUPS = _LATENT // _L


def _sc_permute(target, permutation):
    mesh = plsc.VectorSubcoreMesh(
        core_axis_name="c", subcore_axis_name="s",
        num_cores=_NC, num_subcores=_NS)

    @functools.partial(
        pl.kernel,
        out_type=jax.ShapeDtypeStruct((_BATCH * _LATENT,), jnp.float32),
        mesh=mesh,
        compiler_params=pltpu.CompilerParams(needs_layout_passes=False),
        scratch_types=[
            pltpu.VMEM((_LATENT,), jnp.int32),
            pltpu.VMEM((_CHUNK * _LATENT,), jnp.float32),
            pltpu.VMEM((_CHUNK * _LATENT,), jnp.float32),
        ],
    )
    def body(target_flat, perm_hbm, out_flat, perm_v, inb, outb):
        wid = lax.axis_index("s") * _NC + lax.axis_index("c")
        base = wid * _ROWS_PER_W
        pltpu.sync_copy(perm_hbm, perm_v)
        perm_vecs = [perm_v[pl.ds(g * _L, _L)] for g in range(_GROUPS)]

        def chunk_body(c, carry):
            elem0 = (base + c * _CHUNK) * _LATENT
            pltpu.sync_copy(target_flat.at[pl.ds(elem0, _CHUNK * _LATENT)], inb)

            def row_body(r, idx_vecs):
                for g in range(_GROUPS):
                    vals = plsc.load_gather(inb, [idx_vecs[g]])
                    outb[pl.ds(r * _LATENT + g * _L, _L)] = vals
                return tuple(v + _LATENT for v in idx_vecs)

            lax.fori_loop(0, _CHUNK, row_body, tuple(perm_vecs))
            pltpu.sync_copy(outb, out_flat.at[pl.ds(elem0, _CHUNK * _LATENT)])
            return carry

        lax.fori_loop(0, _NCHUNK, chunk_body, 0)

    flat = body(target.reshape(_BATCH * _LATENT), permutation)
    return flat.reshape(_BATCH, _LATENT)


def kernel(target, permutation):
    return _sc_permute(target, permutation)


# speedup vs baseline: 1.8858x; 1.5084x over previous
import functools

import jax
import jax.numpy as jnp
from jax import lax
from jax.experimental import pallas as pl
from jax.experimental.pallas import tpu as pltpu
from jax.experimental.pallas import tpu_sc as plsc

_LATENT = 128
_BATCH = 16384
_NC = 2
_NS = 16
_L = 16
_NW = _NC * _NS
_ROWS_PER_W = _BATCH // _NW
_CHUNK = 128
_NCHUNK = _ROWS_PER_W // _CHUNK
_CE = _CHUNK * _LATENT
_GROUPS = _LATENT // _L


def _sc_permute(target, permutation):
    mesh = plsc.VectorSubcoreMesh(
        core_axis_name="c", subcore_axis_name="s",
        num_cores=_NC, num_subcores=_NS)

    @functools.partial(
        pl.kernel,
        out_type=jax.ShapeDtypeStruct((_BATCH * _LATENT,), jnp.float32),
        mesh=mesh,
        compiler_params=pltpu.CompilerParams(needs_layout_passes=False),
        scratch_types=[
            pltpu.VMEM((_LATENT,), jnp.int32),
            pltpu.VMEM((_CE,), jnp.float32),
            pltpu.VMEM((_CE,), jnp.float32),
            pltpu.VMEM((_CE,), jnp.float32),
            pltpu.VMEM((_CE,), jnp.float32),
            pltpu.SemaphoreType.DMA,
            pltpu.SemaphoreType.DMA,
            pltpu.SemaphoreType.DMA,
            pltpu.SemaphoreType.DMA,
        ],
    )
    def body(target_flat, perm_hbm, out_flat, perm_v,
             inb0, inb1, outb0, outb1, si0, si1, so0, so1):
        wid = lax.axis_index("s") * _NC + lax.axis_index("c")
        base = wid * _ROWS_PER_W * _LATENT
        pltpu.sync_copy(perm_hbm, perm_v)
        perm_vecs = tuple(perm_v[pl.ds(g * _L, _L)] for g in range(_GROUPS))
        inbs, outbs = [inb0, inb1], [outb0, outb1]
        sis, sos = [si0, si1], [so0, so1]

        in_h = [None, None]
        out_h = [None, None]
        for c in range(min(2, _NCHUNK)):
            in_h[c] = pltpu.async_copy(
                target_flat.at[pl.ds(base + c * _CE, _CE)], inbs[c], sis[c])

        for c in range(_NCHUNK):
            b = c % 2
            in_h[b].wait()
            if out_h[b] is not None:
                out_h[b].wait()
            inb, outb = inbs[b], outbs[b]

            def row_body(r, idxs, inb=inb, outb=outb):
                vals = [plsc.load_gather(inb, [idxs[g]])
                        for g in range(_GROUPS)]
                rb = r * _LATENT
                for g in range(_GROUPS):
                    outb[pl.ds(rb + g * _L, _L)] = vals[g]
                return tuple(v + _LATENT for v in idxs)

            lax.fori_loop(0, _CHUNK, row_body, perm_vecs)
            out_h[b] = pltpu.async_copy(
                outb, out_flat.at[pl.ds(base + c * _CE, _CE)], sos[b])
            if c + 2 < _NCHUNK:
                in_h[b] = pltpu.async_copy(
                    target_flat.at[pl.ds(base + (c + 2) * _CE, _CE)],
                    inbs[b], sis[b])

        for b in range(min(2, _NCHUNK)):
            if out_h[b] is not None:
                out_h[b].wait()

    flat = body(target.reshape(_BATCH * _LATENT), permutation)
    return flat.reshape(_BATCH, _LATENT)


def kernel(target, permutation):
    return _sc_permute(target, permutation)


# speedup vs baseline: 1.9116x; 1.0137x over previous
import functools

import jax
import jax.numpy as jnp
from jax import lax
from jax.experimental import pallas as pl
from jax.experimental.pallas import tpu as pltpu
from jax.experimental.pallas import tpu_sc as plsc

_LATENT = 128
_BATCH = 16384
_NC = 2
_NS = 16
_L = 16
_NW = _NC * _NS
_ROWS_PER_W = _BATCH // _NW
_CHUNK = 128
_NCHUNK = _ROWS_PER_W // _CHUNK
_CE = _CHUNK * _LATENT
_GROUPS = _LATENT // _L


def _sc_permute(target, permutation):
    mesh = plsc.VectorSubcoreMesh(
        core_axis_name="c", subcore_axis_name="s",
        num_cores=_NC, num_subcores=_NS)

    @functools.partial(
        pl.kernel,
        out_type=jax.ShapeDtypeStruct((_BATCH * _LATENT,), jnp.float32),
        mesh=mesh,
        compiler_params=pltpu.CompilerParams(needs_layout_passes=False),
        scratch_types=[
            pltpu.VMEM((_LATENT,), jnp.int32),
            pltpu.VMEM((_CE,), jnp.float32),
            pltpu.VMEM((_CE,), jnp.float32),
            pltpu.VMEM((_CE,), jnp.float32),
            pltpu.VMEM((_CE,), jnp.float32),
            pltpu.SemaphoreType.DMA,
            pltpu.SemaphoreType.DMA,
            pltpu.SemaphoreType.DMA,
            pltpu.SemaphoreType.DMA,
        ],
    )
    def body(target_flat, perm_hbm, out_flat, perm_v,
             inb0, inb1, outb0, outb1, si0, si1, so0, so1):
        wid = lax.axis_index("s") * _NC + lax.axis_index("c")
        base = wid * _ROWS_PER_W * _LATENT
        inbs, outbs = [inb0, inb1], [outb0, outb1]
        sis, sos = [si0, si1], [so0, so1]

        in_h = [None, None]
        out_h = [None, None]
        for c in range(min(2, _NCHUNK)):
            in_h[c] = pltpu.async_copy(
                target_flat.at[pl.ds(base + c * _CE, _CE)], inbs[c], sis[c])
        pltpu.sync_copy(perm_hbm, perm_v)
        perm_vecs = tuple(perm_v[pl.ds(g * _L, _L)] for g in range(_GROUPS))

        for c in range(_NCHUNK):
            b = c % 2
            in_h[b].wait()
            if out_h[b] is not None:
                out_h[b].wait()
            inb, outb = inbs[b], outbs[b]

            def row_body(r, idxs, inb=inb, outb=outb):
                vals = [plsc.load_gather(inb, [idxs[g]])
                        for g in range(_GROUPS)]
                rb = r * _LATENT
                for g in range(_GROUPS):
                    outb[pl.ds(rb + g * _L, _L)] = vals[g]
                return tuple(v + _LATENT for v in idxs)

            lax.fori_loop(0, _CHUNK, row_body, perm_vecs, unroll=2)
            out_h[b] = pltpu.async_copy(
                outb, out_flat.at[pl.ds(base + c * _CE, _CE)], sos[b])
            if c + 2 < _NCHUNK:
                in_h[b] = pltpu.async_copy(
                    target_flat.at[pl.ds(base + (c + 2) * _CE, _CE)],
                    inbs[b], sis[b])

        for b in range(min(2, _NCHUNK)):
            if out_h[b] is not None:
                out_h[b].wait()

    flat = body(target.reshape(_BATCH * _LATENT), permutation)
    return flat.reshape(_BATCH, _LATENT)


def kernel(target, permutation):
    return _sc_permute(target, permutation)
